# baseline (speedup 1.0000x reference)
import jax
import jax.numpy as jnp
from jax.experimental import pallas as pl
from jax.experimental.pallas import tpu as pltpu

_G = 256
_D = 64
_TB = 65536


def _gather_t_kernel(ids_ref, tabt_ref, out_ref):
    pieces = []
    lane_w = _TB // 8
    g = jax.lax.broadcasted_iota(jnp.int32, (_G, 128), 0)
    for t in range(_TB // 128):
        s, l0 = divmod(t * 128, lane_w)
        ids_piece = ids_ref[0, s:s + 1, l0:l0 + 128]
        pieces.append((g == ids_piece).astype(tabt_ref.dtype))
    oh_t = jnp.concatenate(pieces, axis=1)
    out_ref[...] = jnp.dot(tabt_ref[...], oh_t,
                           preferred_element_type=jnp.float32)


def kernel(group_id, table):
    (B,) = group_id.shape
    num_tiles = B // _TB
    ids = group_id.astype(jnp.int32).reshape(num_tiles, 8, _TB // 8)
    tab_t = table[:_G, :_D].T

    out_t = pl.pallas_call(
        _gather_t_kernel,
        out_shape=jax.ShapeDtypeStruct((_D, B), jnp.float32),
        grid=(num_tiles,),
        in_specs=[
            pl.BlockSpec((1, 8, _TB // 8), lambda i: (i, 0, 0)),
            pl.BlockSpec((_D, _G), lambda i: (0, 0)),
        ],
        out_specs=pl.BlockSpec((_D, _TB), lambda i: (0, i)),
        compiler_params=pltpu.CompilerParams(
            dimension_semantics=("parallel",)),
    )(ids, tab_t)

    return out_t.T
